# baseline (speedup 1.0000x reference)
import functools

import jax
import jax.numpy as jnp
from jax import lax
from jax.experimental import pallas as pl
from jax.experimental.pallas import tpu as pltpu
from jax.experimental.pallas import tpu_sc as plsc

NC = 2
NS = 16
NW = NC * NS
CHUNK = 80
NBUF = 4


def _deg_body(n_pad, chunks, dst2_hbm, ones_hbm, zeros_hbm, deg0_hbm,
              deg1_hbm, ones_v, didx_all, deg_sh, zrow_v, sem):
    c = lax.axis_index("c")
    s = lax.axis_index("s")
    wid = s * NC + c
    rows_per_tile = n_pad // NS
    rslice = pl.ds(s * rows_per_tile, rows_per_tile)
    pltpu.sync_copy(zeros_hbm.at[rslice], zrow_v)
    pltpu.sync_copy(zrow_v, deg_sh.at[rslice])
    pltpu.sync_copy(ones_hbm, ones_v)
    pltpu.sync_copy(dst2_hbm.at[pl.ds(wid * chunks, chunks)], didx_all)
    plsc.subcore_barrier()

    def fire(i, carry):
        pltpu.async_copy(ones_v, deg_sh.at[didx_all.at[i]], sem, add=True)
        return carry

    lax.fori_loop(0, chunks, fire, 0, unroll=False)

    def drain(i, carry):
        pltpu.make_async_copy(ones_v, deg_sh.at[didx_all.at[i]], sem).wait()
        return carry

    lax.fori_loop(0, chunks, drain, 0, unroll=False)
    plsc.subcore_barrier()
    pltpu.sync_copy(deg_sh.at[rslice], zrow_v)
    @pl.when(c == 0)
    def _():
        pltpu.sync_copy(zrow_v, deg0_hbm.at[rslice])
    @pl.when(c == 1)
    def _():
        pltpu.sync_copy(zrow_v, deg1_hbm.at[rslice])


SEG_MAX = 16


def _seg_sizes(chunks):
    sizes, rem = [], chunks
    while rem:
        take = min(SEG_MAX, rem)
        sizes.append(take)
        rem -= take
    return sizes


def _agg_body(n_pad, d, chunks, g_hbm, src2_hbm, dst2_hbm, zeros_hbm,
              agg0_hbm, agg1_hbm, sidxA, didxA, sidxB, didxB,
              rows0, rows1, rows2, rows3, agg_sh,
              gsem0, gsem1, gsem2, gsem3, ssem0, ssem1, ssem2, ssem3,
              isem0, isem1):
    c = lax.axis_index("c")
    s = lax.axis_index("s")
    wid = s * NC + c
    rows_per_tile = n_pad // NS
    sizes = _seg_sizes(chunks)
    sbufs = (sidxA, sidxB)
    dbufs = (didxA, didxB)
    rows = (rows0, rows1, rows2, rows3)
    gsems = (gsem0, gsem1, gsem2, gsem3)
    ssems = (ssem0, ssem1, ssem2, ssem3)

    for j in range(rows_per_tile // CHUNK):
        zslice = pl.ds(s * rows_per_tile + j * CHUNK, CHUNK)
        pltpu.sync_copy(zeros_hbm.at[zslice], rows0)
        pltpu.sync_copy(rows0, agg_sh.at[zslice])
    cbase = wid * chunks
    sz0 = sizes[0]
    pltpu.sync_copy(src2_hbm.at[pl.ds(cbase, sz0)], sidxA.at[pl.ds(0, sz0)])
    pltpu.sync_copy(dst2_hbm.at[pl.ds(cbase, sz0)], didxA.at[pl.ds(0, sz0)])
    for b in range(NBUF - 1):
        pltpu.async_copy(g_hbm.at[sidxA.at[b]], rows[b], gsems[b])
    plsc.subcore_barrier()

    def _step(b, gref, sref, gnext=None):
        sx_, ib = gref
        pltpu.make_async_copy(g_hbm.at[sx_.at[ib]], rows[b], gsems[b]).wait()
        dx_, sb = sref
        pltpu.async_copy(rows[b], agg_sh.at[dx_.at[sb]], ssems[b], add=True)
        pb = (b - 1) % NBUF
        pltpu.make_async_copy(rows[pb], agg_sh.at[dx_.at[sb]], ssems[pb]).wait()
        if gnext is not None:
            nx_, nb = gnext
            pltpu.async_copy(g_hbm.at[nx_.at[nb]], rows[pb], gsems[pb])

    off = 0
    for q, sz in enumerate(sizes):
        sx, dx = sbufs[q % 2], dbufs[q % 2]
        sx2, dx2 = sbufs[1 - q % 2], dbufs[1 - q % 2]
        nxt = sizes[q + 1] if q + 1 < len(sizes) else 0
        nquads = sz // NBUF
        if nxt:
            nslice = pl.ds(cbase + off + sz, nxt)
            pltpu.async_copy(src2_hbm.at[nslice], sx2.at[pl.ds(0, nxt)], isem0)
            pltpu.async_copy(dst2_hbm.at[nslice], dx2.at[pl.ds(0, nxt)], isem1)

        first_uniform = 0
        if q == 0:
            for b in range(NBUF):
                pltpu.make_async_copy(g_hbm.at[sx.at[b]], rows[b],
                                      gsems[b]).wait()
                pltpu.async_copy(rows[b], agg_sh.at[dx.at[b]], ssems[b],
                                 add=True)
                pb = (b - 1) % NBUF
                if b > 0:
                    pltpu.make_async_copy(rows[pb], agg_sh.at[dx.at[b]],
                                          ssems[pb]).wait()
                pltpu.async_copy(g_hbm.at[sx.at[b + 3]], rows[pb], gsems[pb])
            first_uniform = 1

        def quad(jj, carry, sx=sx, dx=dx):
            i = jj * NBUF
            for b in range(NBUF):
                _step(b, (sx, i + b), (dx, i + b), (sx, i + b + 3))
            return carry

        if nquads - 1 > first_uniform:
            lax.fori_loop(first_uniform, nquads - 1, quad, 0, unroll=False)

        i = sz - NBUF
        if nxt:
            _step(0, (sx, i), (dx, i), (sx, i + 3))
            pltpu.make_async_copy(src2_hbm.at[pl.ds(cbase, nxt)],
                                  sx2.at[pl.ds(0, nxt)], isem0).wait()
            pltpu.make_async_copy(dst2_hbm.at[pl.ds(cbase, nxt)],
                                  dx2.at[pl.ds(0, nxt)], isem1).wait()
            for b in range(1, NBUF):
                _step(b, (sx, i + b), (dx, i + b), (sx2, b - 1))
        else:
            _step(0, (sx, i), (dx, i), (sx, i + 3))
            for b in range(1, NBUF):
                _step(b, (sx, i + b), (dx, i + b), None)
            pltpu.make_async_copy(rows[NBUF - 1], agg_sh.at[dx.at[sz - 1]],
                                  ssems[NBUF - 1]).wait()
        off += sz

    plsc.subcore_barrier()
    for j in range(rows_per_tile // CHUNK):
        zslice = pl.ds(s * rows_per_tile + j * CHUNK, CHUNK)
        pltpu.sync_copy(agg_sh.at[zslice], rows0)
        @pl.when(c == 0)
        def _():
            pltpu.sync_copy(rows0, agg0_hbm.at[zslice])
        @pl.when(c == 1)
        def _():
            pltpu.sync_copy(rows0, agg1_hbm.at[zslice])


def _mm_body(x_ref, w_ref, h_ref):
    h_ref[...] = jnp.dot(x_ref[...], w_ref[...],
                         preferred_element_type=jnp.float32)


def _scale_body(h_ref, d0_ref, d1_ref, g_ref, dinv_ref):
    deg = d0_ref[...] + d1_ref[...] + 1.0
    di = lax.rsqrt(deg)
    dinv_ref[...] = di
    g_ref[...] = h_ref[...] * di


def _fin_body(a0_ref, a1_ref, g_ref, dinv_ref, b_ref, y_ref):
    acc = a0_ref[...] + a1_ref[...] + g_ref[...]
    y_ref[...] = jnp.maximum(dinv_ref[...] * acc + b_ref[...], 0.0)


def kernel(x, edge_index, W, b):
    n, d = x.shape
    e = edge_index.shape[1]
    n_pad = -(-(n + 1) // (NS * CHUNK)) * (NS * CHUNK)
    chunks = -(-e // (NW * CHUNK))
    chunks = -(-chunks // 8) * 8
    e_pad = NW * chunks * CHUNK

    src = edge_index[0]
    dst = edge_index[1]
    pad_iota = lax.iota(jnp.int32, e_pad - e)
    src_p = jnp.concatenate([src, pad_iota % n])
    dst_p = jnp.concatenate([dst, n + pad_iota % (n_pad - n)])
    x_p = jnp.concatenate([x, jnp.zeros((n_pad - n, d), x.dtype)])
    ones_c = jnp.ones((CHUNK,), jnp.float32)
    zeros_n = jnp.zeros((n_pad,), jnp.float32)
    zeros_nd = jnp.zeros((n_pad, d), jnp.float32)

    mesh = plsc.VectorSubcoreMesh(core_axis_name="c", subcore_axis_name="s")

    deg_k = pl.kernel(
        functools.partial(_deg_body, n_pad, chunks),
        out_type=(jax.ShapeDtypeStruct((n_pad,), jnp.float32),
                  jax.ShapeDtypeStruct((n_pad,), jnp.float32)),
        mesh=mesh,
        scratch_types=[
            pltpu.VMEM((CHUNK,), jnp.float32),
            pltpu.VMEM((chunks, CHUNK), jnp.int32),
            pltpu.VMEM_SHARED((n_pad,), jnp.float32),
            pltpu.VMEM((n_pad // NS,), jnp.float32),
            pltpu.SemaphoreType.DMA,
        ],
    )
    dst2 = dst_p.reshape(NW * chunks, CHUNK)
    deg0, deg1 = deg_k(dst2, ones_c, zeros_n)

    block = n_pad // 8
    grid = n_pad // block
    mm_k = pl.pallas_call(
        _mm_body,
        grid=(grid,),
        in_specs=[
            pl.BlockSpec((block, d), lambda i: (i, 0)),
            pl.BlockSpec((d, d), lambda i: (0, 0)),
        ],
        out_specs=pl.BlockSpec((block, d), lambda i: (i, 0)),
        out_shape=jax.ShapeDtypeStruct((n_pad, d), jnp.float32),
    )
    h = mm_k(x_p, W)

    scale_k = pl.pallas_call(
        _scale_body,
        grid=(grid,),
        in_specs=[
            pl.BlockSpec((block, d), lambda i: (i, 0)),
            pl.BlockSpec((block, 1), lambda i: (i, 0)),
            pl.BlockSpec((block, 1), lambda i: (i, 0)),
        ],
        out_specs=[
            pl.BlockSpec((block, d), lambda i: (i, 0)),
            pl.BlockSpec((block, 1), lambda i: (i, 0)),
        ],
        out_shape=[jax.ShapeDtypeStruct((n_pad, d), jnp.float32),
                   jax.ShapeDtypeStruct((n_pad, 1), jnp.float32)],
    )
    g, dinv = scale_k(h, deg0.reshape(n_pad, 1), deg1.reshape(n_pad, 1))

    agg_k = pl.kernel(
        functools.partial(_agg_body, n_pad, d, chunks),
        out_type=(jax.ShapeDtypeStruct((n_pad, d), jnp.float32),
                  jax.ShapeDtypeStruct((n_pad, d), jnp.float32)),
        mesh=mesh,
        scratch_types=[
            pltpu.VMEM((min(chunks, SEG_MAX), CHUNK), jnp.int32),
            pltpu.VMEM((min(chunks, SEG_MAX), CHUNK), jnp.int32),
            pltpu.VMEM((min(chunks, SEG_MAX), CHUNK), jnp.int32),
            pltpu.VMEM((min(chunks, SEG_MAX), CHUNK), jnp.int32),
            pltpu.VMEM((CHUNK, d), jnp.float32),
            pltpu.VMEM((CHUNK, d), jnp.float32),
            pltpu.VMEM((CHUNK, d), jnp.float32),
            pltpu.VMEM((CHUNK, d), jnp.float32),
            pltpu.VMEM_SHARED((n_pad, d), jnp.float32),
        ] + [pltpu.SemaphoreType.DMA] * 10,
    )
    agg0, agg1 = agg_k(g, src_p.reshape(NW * chunks, CHUNK), dst2, zeros_nd)

    fin_k = pl.pallas_call(
        _fin_body,
        grid=(grid,),
        in_specs=[
            pl.BlockSpec((block, d), lambda i: (i, 0)),
            pl.BlockSpec((block, d), lambda i: (i, 0)),
            pl.BlockSpec((block, d), lambda i: (i, 0)),
            pl.BlockSpec((block, 1), lambda i: (i, 0)),
            pl.BlockSpec((1, d), lambda i: (0, 0)),
        ],
        out_specs=pl.BlockSpec((block, d), lambda i: (i, 0)),
        out_shape=jax.ShapeDtypeStruct((n_pad, d), jnp.float32),
    )
    y = fin_k(agg0, agg1, g, dinv, b.reshape(1, d))
    return y[:n]


# speedup vs baseline: 1.0607x; 1.0607x over previous
import functools

import jax
import jax.numpy as jnp
from jax import lax
from jax.experimental import pallas as pl
from jax.experimental.pallas import tpu as pltpu
from jax.experimental.pallas import tpu_sc as plsc

NC = 2
NS = 16
NW = NC * NS
CHUNK = 128


def _deg_body(n_pad, chunks, dst2_hbm, ones_hbm, zeros_hbm, deg0_hbm,
              deg1_hbm, ones_v, didx_all, deg_sh, zrow_v, sem):
    c = lax.axis_index("c")
    s = lax.axis_index("s")
    wid = s * NC + c
    rows_per_tile = n_pad // NS
    rslice = pl.ds(s * rows_per_tile, rows_per_tile)
    pltpu.sync_copy(zeros_hbm.at[rslice], zrow_v)
    pltpu.sync_copy(zrow_v, deg_sh.at[rslice])
    pltpu.sync_copy(ones_hbm, ones_v)
    pltpu.sync_copy(dst2_hbm.at[pl.ds(wid * chunks, chunks)], didx_all)
    plsc.subcore_barrier()

    def fire(i, carry):
        pltpu.async_copy(ones_v, deg_sh.at[didx_all.at[i]], sem, add=True)
        return carry

    lax.fori_loop(0, chunks, fire, 0, unroll=False)

    def drain(i, carry):
        pltpu.make_async_copy(ones_v, deg_sh.at[didx_all.at[i]], sem).wait()
        return carry

    lax.fori_loop(0, chunks, drain, 0, unroll=False)
    plsc.subcore_barrier()
    pltpu.sync_copy(deg_sh.at[rslice], zrow_v)
    @pl.when(c == 0)
    def _():
        pltpu.sync_copy(zrow_v, deg0_hbm.at[rslice])
    @pl.when(c == 1)
    def _():
        pltpu.sync_copy(zrow_v, deg1_hbm.at[rslice])


SEG_MAX = 24


def _seg_sizes(chunks):
    sizes, rem = [], chunks
    while rem:
        take = min(SEG_MAX, rem)
        sizes.append(take)
        rem -= take
    return sizes


def _agg_body(n_pad, d, chunks, g_hbm, src2_hbm, dst2_hbm, zeros_hbm,
              agg0_hbm, agg1_hbm, sidxA, didxA, sidxB, didxB, rows0, rows1,
              agg_sh, sem0, sem1, isem0, isem1):
    c = lax.axis_index("c")
    s = lax.axis_index("s")
    wid = s * NC + c
    rows_per_tile = n_pad // NS
    sizes = _seg_sizes(chunks)
    sbufs = (sidxA, sidxB)
    dbufs = (didxA, didxB)
    for j in range(rows_per_tile // CHUNK):
        zslice = pl.ds(s * rows_per_tile + j * CHUNK, CHUNK)
        pltpu.sync_copy(zeros_hbm.at[zslice], rows0)
        pltpu.sync_copy(rows0, agg_sh.at[zslice])
    cbase = wid * chunks
    sz0 = sizes[0]
    pltpu.sync_copy(src2_hbm.at[pl.ds(cbase, sz0)], sidxA.at[pl.ds(0, sz0)])
    pltpu.sync_copy(dst2_hbm.at[pl.ds(cbase, sz0)], didxA.at[pl.ds(0, sz0)])
    plsc.subcore_barrier()

    pltpu.async_copy(g_hbm.at[sidxA.at[0]], rows0, sem0)
    pltpu.async_copy(g_hbm.at[sidxA.at[1]], rows1, sem1)

    off = 0
    for q, sz in enumerate(sizes):
        sx, dx = sbufs[q % 2], dbufs[q % 2]
        sx2, dx2 = sbufs[1 - q % 2], dbufs[1 - q % 2]
        nxt = sizes[q + 1] if q + 1 < len(sizes) else 0
        if nxt:
            nslice = pl.ds(cbase + off + sz, nxt)
            pltpu.async_copy(src2_hbm.at[nslice], sx2.at[pl.ds(0, nxt)], isem0)
            pltpu.async_copy(dst2_hbm.at[nslice], dx2.at[pl.ds(0, nxt)], isem1)

        def step(j, carry, sx=sx, dx=dx):
            i = 2 * j
            pltpu.make_async_copy(g_hbm.at[sx.at[i]], rows0, sem0).wait()
            pltpu.sync_copy(rows0, agg_sh.at[dx.at[i]], add=True)
            pltpu.async_copy(g_hbm.at[sx.at[i + 2]], rows0, sem0)
            pltpu.make_async_copy(g_hbm.at[sx.at[i + 1]], rows1, sem1).wait()
            pltpu.sync_copy(rows1, agg_sh.at[dx.at[i + 1]], add=True)
            pltpu.async_copy(g_hbm.at[sx.at[i + 3]], rows1, sem1)
            return carry

        lax.fori_loop(0, sz // 2 - 1, step, 0, unroll=False)
        last = sz - 2
        pltpu.make_async_copy(g_hbm.at[sx.at[last]], rows0, sem0).wait()
        pltpu.sync_copy(rows0, agg_sh.at[dx.at[last]], add=True)
        if nxt:
            pltpu.make_async_copy(src2_hbm.at[pl.ds(cbase, nxt)],
                                  sx2.at[pl.ds(0, nxt)], isem0).wait()
            pltpu.async_copy(g_hbm.at[sx2.at[0]], rows0, sem0)
        pltpu.make_async_copy(g_hbm.at[sx.at[last + 1]], rows1, sem1).wait()
        pltpu.sync_copy(rows1, agg_sh.at[dx.at[last + 1]], add=True)
        if nxt:
            pltpu.make_async_copy(dst2_hbm.at[pl.ds(cbase, nxt)],
                                  dx2.at[pl.ds(0, nxt)], isem1).wait()
            pltpu.async_copy(g_hbm.at[sx2.at[1]], rows1, sem1)
        off += sz

    plsc.subcore_barrier()
    for j in range(rows_per_tile // CHUNK):
        zslice = pl.ds(s * rows_per_tile + j * CHUNK, CHUNK)
        pltpu.sync_copy(agg_sh.at[zslice], rows0)
        @pl.when(c == 0)
        def _():
            pltpu.sync_copy(rows0, agg0_hbm.at[zslice])
        @pl.when(c == 1)
        def _():
            pltpu.sync_copy(rows0, agg1_hbm.at[zslice])


def _mm_body(x_ref, w_ref, d0_ref, d1_ref, g_ref, dinv_ref):
    deg = d0_ref[...] + d1_ref[...] + 1.0
    di = lax.rsqrt(deg)
    dinv_ref[...] = di
    g_ref[...] = jnp.dot(x_ref[...], w_ref[...],
                         preferred_element_type=jnp.float32) * di


def _fin_body(a0_ref, a1_ref, g_ref, dinv_ref, b_ref, y_ref):
    acc = a0_ref[...] + a1_ref[...] + g_ref[...]
    y_ref[...] = jnp.maximum(dinv_ref[...] * acc + b_ref[...], 0.0)


def kernel(x, edge_index, W, b):
    n, d = x.shape
    e = edge_index.shape[1]
    n_pad = -(-(n + 1) // (NS * CHUNK)) * (NS * CHUNK)
    chunks = -(-e // (NW * CHUNK))
    chunks = -(-chunks // 8) * 8
    e_pad = NW * chunks * CHUNK

    src = edge_index[0]
    dst = edge_index[1]
    pad_iota = lax.iota(jnp.int32, e_pad - e)
    src_p = jnp.concatenate([src, pad_iota % n])
    dst_p = jnp.concatenate([dst, n + pad_iota % (n_pad - n)])
    ones_c = jnp.ones((CHUNK,), jnp.float32)
    zeros_n = jnp.zeros((n_pad,), jnp.float32)
    zeros_nd = jnp.zeros((n_pad, d), jnp.float32)

    mesh = plsc.VectorSubcoreMesh(core_axis_name="c", subcore_axis_name="s")

    deg_k = pl.kernel(
        functools.partial(_deg_body, n_pad, chunks),
        out_type=(jax.ShapeDtypeStruct((n_pad,), jnp.float32),
                  jax.ShapeDtypeStruct((n_pad,), jnp.float32)),
        mesh=mesh,
        scratch_types=[
            pltpu.VMEM((CHUNK,), jnp.float32),
            pltpu.VMEM((chunks, CHUNK), jnp.int32),
            pltpu.VMEM_SHARED((n_pad,), jnp.float32),
            pltpu.VMEM((n_pad // NS,), jnp.float32),
            pltpu.SemaphoreType.DMA,
        ],
    )
    dst2 = dst_p.reshape(NW * chunks, CHUNK)
    deg0, deg1 = deg_k(dst2, ones_c, zeros_n)

    grid = next(gg for gg in (8, 10, 16, 20, 25, 40, 50, 125, n)
                if n % gg == 0 and (n // gg) % 8 == 0)
    block = n // grid
    mm_k = pl.pallas_call(
        _mm_body,
        grid=(grid,),
        in_specs=[
            pl.BlockSpec((block, d), lambda i: (i, 0)),
            pl.BlockSpec((d, d), lambda i: (0, 0)),
            pl.BlockSpec((block, 1), lambda i: (i, 0)),
            pl.BlockSpec((block, 1), lambda i: (i, 0)),
        ],
        out_specs=[
            pl.BlockSpec((block, d), lambda i: (i, 0)),
            pl.BlockSpec((block, 1), lambda i: (i, 0)),
        ],
        out_shape=[jax.ShapeDtypeStruct((n, d), jnp.float32),
                   jax.ShapeDtypeStruct((n, 1), jnp.float32)],
    )
    g, dinv = mm_k(x, W, deg0.reshape(n_pad, 1), deg1.reshape(n_pad, 1))

    agg_k = pl.kernel(
        functools.partial(_agg_body, n_pad, d, chunks),
        out_type=(jax.ShapeDtypeStruct((n_pad, d), jnp.float32),
                  jax.ShapeDtypeStruct((n_pad, d), jnp.float32)),
        mesh=mesh,
        scratch_types=[
            pltpu.VMEM((min(chunks, SEG_MAX), CHUNK), jnp.int32),
            pltpu.VMEM((min(chunks, SEG_MAX), CHUNK), jnp.int32),
            pltpu.VMEM((min(chunks, SEG_MAX), CHUNK), jnp.int32),
            pltpu.VMEM((min(chunks, SEG_MAX), CHUNK), jnp.int32),
            pltpu.VMEM((CHUNK, d), jnp.float32),
            pltpu.VMEM((CHUNK, d), jnp.float32),
            pltpu.VMEM_SHARED((n_pad, d), jnp.float32),
            pltpu.SemaphoreType.DMA,
            pltpu.SemaphoreType.DMA,
            pltpu.SemaphoreType.DMA,
            pltpu.SemaphoreType.DMA,
        ],
    )
    agg0, agg1 = agg_k(g, src_p.reshape(NW * chunks, CHUNK), dst2, zeros_nd)

    fin_k = pl.pallas_call(
        _fin_body,
        grid=(grid,),
        in_specs=[
            pl.BlockSpec((block, d), lambda i: (i, 0)),
            pl.BlockSpec((block, d), lambda i: (i, 0)),
            pl.BlockSpec((block, d), lambda i: (i, 0)),
            pl.BlockSpec((block, 1), lambda i: (i, 0)),
            pl.BlockSpec((1, d), lambda i: (0, 0)),
        ],
        out_specs=pl.BlockSpec((block, d), lambda i: (i, 0)),
        out_shape=jax.ShapeDtypeStruct((n, d), jnp.float32),
    )
    return fin_k(agg0, agg1, g, dinv, b.reshape(1, d))


# speedup vs baseline: 1.0895x; 1.0271x over previous
import functools

import jax
import jax.numpy as jnp
from jax import lax
from jax.experimental import pallas as pl
from jax.experimental.pallas import tpu as pltpu
from jax.experimental.pallas import tpu_sc as plsc

NC = 2
NS = 16
NW = NC * NS
CHUNK = 128


def _deg_body(n_pad, chunks, dst2_hbm, ones_hbm, zeros_hbm, deg0_hbm,
              deg1_hbm, ones_v, didx_all, deg_sh, zrow_v, sem):
    c = lax.axis_index("c")
    s = lax.axis_index("s")
    wid = s * NC + c
    rows_per_tile = n_pad // NS
    rslice = pl.ds(s * rows_per_tile, rows_per_tile)
    pltpu.sync_copy(zeros_hbm.at[rslice], zrow_v)
    pltpu.sync_copy(zrow_v, deg_sh.at[rslice])
    pltpu.sync_copy(ones_hbm, ones_v)
    pltpu.sync_copy(dst2_hbm.at[pl.ds(wid * chunks, chunks)], didx_all)
    plsc.subcore_barrier()

    def fire(i, carry):
        pltpu.async_copy(ones_v, deg_sh.at[didx_all.at[i]], sem, add=True)
        return carry

    lax.fori_loop(0, chunks, fire, 0, unroll=False)

    def drain(i, carry):
        pltpu.make_async_copy(ones_v, deg_sh.at[didx_all.at[i]], sem).wait()
        return carry

    lax.fori_loop(0, chunks, drain, 0, unroll=False)
    plsc.subcore_barrier()
    pltpu.sync_copy(deg_sh.at[rslice], zrow_v)
    @pl.when(c == 0)
    def _():
        pltpu.sync_copy(zrow_v, deg0_hbm.at[rslice])
    @pl.when(c == 1)
    def _():
        pltpu.sync_copy(zrow_v, deg1_hbm.at[rslice])


SEG_MAX = 24


def _seg_sizes(chunks):
    sizes, rem = [], chunks
    while rem:
        take = min(SEG_MAX, rem)
        sizes.append(take)
        rem -= take
    return sizes


def _agg_body(n_pad, d, chunks, g_hbm, src2_hbm, dst2_hbm, zeros_hbm,
              agg0_hbm, agg1_hbm, sidxA, didxA, sidxB, didxB, rows0, rows1,
              agg_sh, sem0, sem1, isem0, isem1):
    c = lax.axis_index("c")
    s = lax.axis_index("s")
    wid = s * NC + c
    rows_per_tile = n_pad // NS
    sizes = _seg_sizes(chunks)
    sbufs = (sidxA, sidxB)
    dbufs = (didxA, didxB)
    pltpu.sync_copy(zeros_hbm, rows0)
    for j in range(rows_per_tile // CHUNK):
        zslice = pl.ds(s * rows_per_tile + j * CHUNK, CHUNK)
        pltpu.sync_copy(rows0, agg_sh.at[zslice])
    cbase = wid * chunks
    sz0 = sizes[0]
    pltpu.sync_copy(src2_hbm.at[pl.ds(cbase, sz0)], sidxA.at[pl.ds(0, sz0)])
    pltpu.sync_copy(dst2_hbm.at[pl.ds(cbase, sz0)], didxA.at[pl.ds(0, sz0)])
    plsc.subcore_barrier()

    pltpu.async_copy(g_hbm.at[sidxA.at[0]], rows0, sem0)
    pltpu.async_copy(g_hbm.at[sidxA.at[1]], rows1, sem1)

    off = 0
    for q, sz in enumerate(sizes):
        sx, dx = sbufs[q % 2], dbufs[q % 2]
        sx2, dx2 = sbufs[1 - q % 2], dbufs[1 - q % 2]
        nxt = sizes[q + 1] if q + 1 < len(sizes) else 0
        if nxt:
            nslice = pl.ds(cbase + off + sz, nxt)
            pltpu.async_copy(src2_hbm.at[nslice], sx2.at[pl.ds(0, nxt)], isem0)
            pltpu.async_copy(dst2_hbm.at[nslice], dx2.at[pl.ds(0, nxt)], isem1)

        def step(j, carry, sx=sx, dx=dx):
            i = 2 * j
            pltpu.make_async_copy(g_hbm.at[sx.at[i]], rows0, sem0).wait()
            pltpu.sync_copy(rows0, agg_sh.at[dx.at[i]], add=True)
            pltpu.async_copy(g_hbm.at[sx.at[i + 2]], rows0, sem0)
            pltpu.make_async_copy(g_hbm.at[sx.at[i + 1]], rows1, sem1).wait()
            pltpu.sync_copy(rows1, agg_sh.at[dx.at[i + 1]], add=True)
            pltpu.async_copy(g_hbm.at[sx.at[i + 3]], rows1, sem1)
            return carry

        lax.fori_loop(0, sz // 2 - 1, step, 0, unroll=False)
        last = sz - 2
        pltpu.make_async_copy(g_hbm.at[sx.at[last]], rows0, sem0).wait()
        pltpu.sync_copy(rows0, agg_sh.at[dx.at[last]], add=True)
        if nxt:
            pltpu.make_async_copy(src2_hbm.at[pl.ds(cbase, nxt)],
                                  sx2.at[pl.ds(0, nxt)], isem0).wait()
            pltpu.async_copy(g_hbm.at[sx2.at[0]], rows0, sem0)
        pltpu.make_async_copy(g_hbm.at[sx.at[last + 1]], rows1, sem1).wait()
        pltpu.sync_copy(rows1, agg_sh.at[dx.at[last + 1]], add=True)
        if nxt:
            pltpu.make_async_copy(dst2_hbm.at[pl.ds(cbase, nxt)],
                                  dx2.at[pl.ds(0, nxt)], isem1).wait()
            pltpu.async_copy(g_hbm.at[sx2.at[1]], rows1, sem1)
        off += sz

    plsc.subcore_barrier()
    rslice = pl.ds(s * rows_per_tile, rows_per_tile)
    @pl.when(c == 0)
    def _():
        pltpu.sync_copy(agg_sh.at[rslice], agg0_hbm.at[rslice])
    @pl.when(c == 1)
    def _():
        pltpu.sync_copy(agg_sh.at[rslice], agg1_hbm.at[rslice])


def _mm_body(x_ref, w_ref, d0_ref, d1_ref, g_ref, dinv_ref):
    deg = d0_ref[...] + d1_ref[...] + 1.0
    di = lax.rsqrt(deg)
    dinv_ref[...] = di
    g_ref[...] = jnp.dot(x_ref[...], w_ref[...],
                         preferred_element_type=jnp.float32) * di


def _fin_body(a0_ref, a1_ref, g_ref, dinv_ref, b_ref, y_ref):
    acc = a0_ref[...] + a1_ref[...] + g_ref[...]
    y_ref[...] = jnp.maximum(dinv_ref[...] * acc + b_ref[...], 0.0)


def kernel(x, edge_index, W, b):
    n, d = x.shape
    e = edge_index.shape[1]
    n_pad = -(-(n + 1) // (NS * CHUNK)) * (NS * CHUNK)
    chunks = -(-e // (NW * CHUNK))
    chunks = -(-chunks // 8) * 8
    e_pad = NW * chunks * CHUNK

    src = edge_index[0]
    dst = edge_index[1]
    pad_iota = lax.iota(jnp.int32, e_pad - e)
    src_p = jnp.concatenate([src, pad_iota % n])
    dst_p = jnp.concatenate([dst, n + pad_iota % (n_pad - n)])
    ones_c = jnp.ones((CHUNK,), jnp.float32)
    zeros_n = jnp.zeros((n_pad,), jnp.float32)
    zeros_nd = jnp.zeros((CHUNK, d), jnp.float32)

    mesh = plsc.VectorSubcoreMesh(core_axis_name="c", subcore_axis_name="s")

    deg_k = pl.kernel(
        functools.partial(_deg_body, n_pad, chunks),
        out_type=(jax.ShapeDtypeStruct((n_pad,), jnp.float32),
                  jax.ShapeDtypeStruct((n_pad,), jnp.float32)),
        mesh=mesh,
        scratch_types=[
            pltpu.VMEM((CHUNK,), jnp.float32),
            pltpu.VMEM((chunks, CHUNK), jnp.int32),
            pltpu.VMEM_SHARED((n_pad,), jnp.float32),
            pltpu.VMEM((n_pad // NS,), jnp.float32),
            pltpu.SemaphoreType.DMA,
        ],
    )
    dst2 = dst_p.reshape(NW * chunks, CHUNK)
    deg0, deg1 = deg_k(dst2, ones_c, zeros_n)

    grid = next(gg for gg in (8, 10, 16, 20, 25, 40, 50, 125, n)
                if n % gg == 0 and (n // gg) % 8 == 0)
    block = n // grid
    mm_k = pl.pallas_call(
        _mm_body,
        grid=(grid,),
        in_specs=[
            pl.BlockSpec((block, d), lambda i: (i, 0)),
            pl.BlockSpec((d, d), lambda i: (0, 0)),
            pl.BlockSpec((block, 1), lambda i: (i, 0)),
            pl.BlockSpec((block, 1), lambda i: (i, 0)),
        ],
        out_specs=[
            pl.BlockSpec((block, d), lambda i: (i, 0)),
            pl.BlockSpec((block, 1), lambda i: (i, 0)),
        ],
        out_shape=[jax.ShapeDtypeStruct((n, d), jnp.float32),
                   jax.ShapeDtypeStruct((n, 1), jnp.float32)],
    )
    g, dinv = mm_k(x, W, deg0.reshape(n_pad, 1), deg1.reshape(n_pad, 1))

    agg_k = pl.kernel(
        functools.partial(_agg_body, n_pad, d, chunks),
        out_type=(jax.ShapeDtypeStruct((n_pad, d), jnp.float32),
                  jax.ShapeDtypeStruct((n_pad, d), jnp.float32)),
        mesh=mesh,
        scratch_types=[
            pltpu.VMEM((min(chunks, SEG_MAX), CHUNK), jnp.int32),
            pltpu.VMEM((min(chunks, SEG_MAX), CHUNK), jnp.int32),
            pltpu.VMEM((min(chunks, SEG_MAX), CHUNK), jnp.int32),
            pltpu.VMEM((min(chunks, SEG_MAX), CHUNK), jnp.int32),
            pltpu.VMEM((CHUNK, d), jnp.float32),
            pltpu.VMEM((CHUNK, d), jnp.float32),
            pltpu.VMEM_SHARED((n_pad, d), jnp.float32),
            pltpu.SemaphoreType.DMA,
            pltpu.SemaphoreType.DMA,
            pltpu.SemaphoreType.DMA,
            pltpu.SemaphoreType.DMA,
        ],
    )
    agg0, agg1 = agg_k(g, src_p.reshape(NW * chunks, CHUNK), dst2, zeros_nd)

    fin_k = pl.pallas_call(
        _fin_body,
        grid=(grid,),
        in_specs=[
            pl.BlockSpec((block, d), lambda i: (i, 0)),
            pl.BlockSpec((block, d), lambda i: (i, 0)),
            pl.BlockSpec((block, d), lambda i: (i, 0)),
            pl.BlockSpec((block, 1), lambda i: (i, 0)),
            pl.BlockSpec((1, d), lambda i: (0, 0)),
        ],
        out_specs=pl.BlockSpec((block, d), lambda i: (i, 0)),
        out_shape=jax.ShapeDtypeStruct((n, d), jnp.float32),
    )
    return fin_k(agg0, agg1, g, dinv, b.reshape(1, d))


# speedup vs baseline: 1.0968x; 1.0067x over previous
import functools

import jax
import jax.numpy as jnp
import numpy as np
from jax import lax
from jax.experimental import pallas as pl
from jax.experimental.pallas import tpu as pltpu
from jax.experimental.pallas import tpu_sc as plsc

NC = 2
NS = 16
NW = NC * NS
CHUNK = 128


def _deg_body(n_pad, chunks, dst2_hbm, ones_hbm, zeros_hbm, deg0_hbm,
              deg1_hbm, ones_v, didx_all, deg_sh, zrow_v, sem):
    c = lax.axis_index("c")
    s = lax.axis_index("s")
    wid = s * NC + c
    rows_per_tile = n_pad // NS
    rslice = pl.ds(s * rows_per_tile, rows_per_tile)
    pltpu.sync_copy(zeros_hbm.at[rslice], zrow_v)
    pltpu.sync_copy(zrow_v, deg_sh.at[rslice])
    pltpu.sync_copy(ones_hbm, ones_v)
    pltpu.sync_copy(dst2_hbm.at[pl.ds(wid * chunks, chunks)], didx_all)
    plsc.subcore_barrier()

    def fire(i, carry):
        pltpu.async_copy(ones_v, deg_sh.at[didx_all.at[i]], sem, add=True)
        return carry

    lax.fori_loop(0, chunks, fire, 0, unroll=False)

    def drain(i, carry):
        pltpu.make_async_copy(ones_v, deg_sh.at[didx_all.at[i]], sem).wait()
        return carry

    lax.fori_loop(0, chunks, drain, 0, unroll=False)
    plsc.subcore_barrier()
    pltpu.sync_copy(deg_sh.at[rslice], zrow_v)
    @pl.when(c == 0)
    def _():
        pltpu.sync_copy(zrow_v, deg0_hbm.at[rslice])
    @pl.when(c == 1)
    def _():
        pltpu.sync_copy(zrow_v, deg1_hbm.at[rslice])


SEG_MAX = 24


def _seg_sizes(chunks):
    sizes, rem = [], chunks
    while rem:
        take = min(SEG_MAX, rem)
        sizes.append(take)
        rem -= take
    return sizes


def _agg_body(n_pad, d, chunks, g_hbm, src2_hbm, dst2_hbm, zeros_hbm,
              agg0_hbm, agg1_hbm, sidxA, didxA, sidxB, didxB, rows0, rows1,
              agg_sh, sem0, sem1, isem0, isem1):
    c = lax.axis_index("c")
    s = lax.axis_index("s")
    wid = s * NC + c
    rows_per_tile = n_pad // NS
    sizes = _seg_sizes(chunks)
    sbufs = (sidxA, sidxB)
    dbufs = (didxA, didxB)
    pltpu.sync_copy(zeros_hbm, rows0)
    for j in range(rows_per_tile // CHUNK):
        zslice = pl.ds(s * rows_per_tile + j * CHUNK, CHUNK)
        pltpu.sync_copy(rows0, agg_sh.at[zslice])
    cbase = wid * chunks
    sz0 = sizes[0]
    pltpu.sync_copy(src2_hbm.at[pl.ds(cbase, sz0)], sidxA.at[pl.ds(0, sz0)])
    pltpu.sync_copy(dst2_hbm.at[pl.ds(cbase, sz0)], didxA.at[pl.ds(0, sz0)])
    plsc.subcore_barrier()

    pltpu.async_copy(g_hbm.at[sidxA.at[0]], rows0, sem0)
    pltpu.async_copy(g_hbm.at[sidxA.at[1]], rows1, sem1)

    off = 0
    for q, sz in enumerate(sizes):
        sx, dx = sbufs[q % 2], dbufs[q % 2]
        sx2, dx2 = sbufs[1 - q % 2], dbufs[1 - q % 2]
        nxt = sizes[q + 1] if q + 1 < len(sizes) else 0
        if nxt:
            nslice = pl.ds(cbase + off + sz, nxt)
            pltpu.async_copy(src2_hbm.at[nslice], sx2.at[pl.ds(0, nxt)], isem0)
            pltpu.async_copy(dst2_hbm.at[nslice], dx2.at[pl.ds(0, nxt)], isem1)

        def step(j, carry, sx=sx, dx=dx):
            i = 2 * j
            pltpu.make_async_copy(g_hbm.at[sx.at[i]], rows0, sem0).wait()
            pltpu.sync_copy(rows0, agg_sh.at[dx.at[i]], add=True)
            pltpu.async_copy(g_hbm.at[sx.at[i + 2]], rows0, sem0)
            pltpu.make_async_copy(g_hbm.at[sx.at[i + 1]], rows1, sem1).wait()
            pltpu.sync_copy(rows1, agg_sh.at[dx.at[i + 1]], add=True)
            pltpu.async_copy(g_hbm.at[sx.at[i + 3]], rows1, sem1)
            return carry

        lax.fori_loop(0, sz // 2 - 1, step, 0, unroll=False)
        last = sz - 2
        pltpu.make_async_copy(g_hbm.at[sx.at[last]], rows0, sem0).wait()
        pltpu.sync_copy(rows0, agg_sh.at[dx.at[last]], add=True)
        if nxt:
            pltpu.make_async_copy(src2_hbm.at[pl.ds(cbase, nxt)],
                                  sx2.at[pl.ds(0, nxt)], isem0).wait()
            pltpu.async_copy(g_hbm.at[sx2.at[0]], rows0, sem0)
        pltpu.make_async_copy(g_hbm.at[sx.at[last + 1]], rows1, sem1).wait()
        pltpu.sync_copy(rows1, agg_sh.at[dx.at[last + 1]], add=True)
        if nxt:
            pltpu.make_async_copy(dst2_hbm.at[pl.ds(cbase, nxt)],
                                  dx2.at[pl.ds(0, nxt)], isem1).wait()
            pltpu.async_copy(g_hbm.at[sx2.at[1]], rows1, sem1)
        off += sz

    plsc.subcore_barrier()
    rslice = pl.ds(s * rows_per_tile, rows_per_tile)
    @pl.when(c == 0)
    def _():
        pltpu.sync_copy(agg_sh.at[rslice], agg0_hbm.at[rslice])
    @pl.when(c == 1)
    def _():
        pltpu.sync_copy(agg_sh.at[rslice], agg1_hbm.at[rslice])


def _mm_body(x_ref, w_ref, d0_ref, d1_ref, g_ref, dinv_ref):
    deg = d0_ref[...] + d1_ref[...] + 1.0
    di = lax.rsqrt(deg)
    dinv_ref[...] = di
    g_ref[...] = jnp.dot(x_ref[...], w_ref[...],
                         preferred_element_type=jnp.float32) * di


def _fin_body(a0_ref, a1_ref, g_ref, dinv_ref, b_ref, y_ref):
    acc = a0_ref[...] + a1_ref[...] + g_ref[...]
    y_ref[...] = jnp.maximum(dinv_ref[...] * acc + b_ref[...], 0.0)


def kernel(x, edge_index, W, b):
    n, d = x.shape
    e = edge_index.shape[1]
    n_pad = -(-(n + 1) // (NS * CHUNK)) * (NS * CHUNK)
    chunks = -(-e // (NW * CHUNK))
    chunks = -(-chunks // 8) * 8
    e_pad = NW * chunks * CHUNK

    src = edge_index[0]
    dst = edge_index[1]
    pad_iota = np.arange(e_pad - e, dtype=np.int32)
    src_p = jnp.concatenate([src, jnp.asarray(pad_iota % n)])
    dst_p = jnp.concatenate([dst, jnp.asarray(n + pad_iota % (n_pad - n))])
    ones_c = jnp.ones((CHUNK,), jnp.float32)
    zeros_n = jnp.zeros((n_pad,), jnp.float32)
    zeros_nd = jnp.zeros((CHUNK, d), jnp.float32)

    mesh = plsc.VectorSubcoreMesh(core_axis_name="c", subcore_axis_name="s")

    deg_k = pl.kernel(
        functools.partial(_deg_body, n_pad, chunks),
        out_type=(jax.ShapeDtypeStruct((n_pad,), jnp.float32),
                  jax.ShapeDtypeStruct((n_pad,), jnp.float32)),
        mesh=mesh,
        scratch_types=[
            pltpu.VMEM((CHUNK,), jnp.float32),
            pltpu.VMEM((chunks, CHUNK), jnp.int32),
            pltpu.VMEM_SHARED((n_pad,), jnp.float32),
            pltpu.VMEM((n_pad // NS,), jnp.float32),
            pltpu.SemaphoreType.DMA,
        ],
    )
    dst2 = dst_p.reshape(NW * chunks, CHUNK)
    deg0, deg1 = deg_k(dst2, ones_c, zeros_n)

    grid = next(gg for gg in (8, 10, 16, 20, 25, 40, 50, 125, n)
                if n % gg == 0 and (n // gg) % 8 == 0)
    block = n // grid
    mm_k = pl.pallas_call(
        _mm_body,
        grid=(grid,),
        in_specs=[
            pl.BlockSpec((block, d), lambda i: (i, 0)),
            pl.BlockSpec((d, d), lambda i: (0, 0)),
            pl.BlockSpec((block, 1), lambda i: (i, 0)),
            pl.BlockSpec((block, 1), lambda i: (i, 0)),
        ],
        out_specs=[
            pl.BlockSpec((block, d), lambda i: (i, 0)),
            pl.BlockSpec((block, 1), lambda i: (i, 0)),
        ],
        out_shape=[jax.ShapeDtypeStruct((n, d), jnp.float32),
                   jax.ShapeDtypeStruct((n, 1), jnp.float32)],
    )
    g, dinv = mm_k(x, W, deg0.reshape(n_pad, 1), deg1.reshape(n_pad, 1))

    agg_k = pl.kernel(
        functools.partial(_agg_body, n_pad, d, chunks),
        out_type=(jax.ShapeDtypeStruct((n_pad, d), jnp.float32),
                  jax.ShapeDtypeStruct((n_pad, d), jnp.float32)),
        mesh=mesh,
        scratch_types=[
            pltpu.VMEM((min(chunks, SEG_MAX), CHUNK), jnp.int32),
            pltpu.VMEM((min(chunks, SEG_MAX), CHUNK), jnp.int32),
            pltpu.VMEM((min(chunks, SEG_MAX), CHUNK), jnp.int32),
            pltpu.VMEM((min(chunks, SEG_MAX), CHUNK), jnp.int32),
            pltpu.VMEM((CHUNK, d), jnp.float32),
            pltpu.VMEM((CHUNK, d), jnp.float32),
            pltpu.VMEM_SHARED((n_pad, d), jnp.float32),
            pltpu.SemaphoreType.DMA,
            pltpu.SemaphoreType.DMA,
            pltpu.SemaphoreType.DMA,
            pltpu.SemaphoreType.DMA,
        ],
    )
    agg0, agg1 = agg_k(g, src_p.reshape(NW * chunks, CHUNK), dst2, zeros_nd)

    fin_k = pl.pallas_call(
        _fin_body,
        grid=(grid,),
        in_specs=[
            pl.BlockSpec((block, d), lambda i: (i, 0)),
            pl.BlockSpec((block, d), lambda i: (i, 0)),
            pl.BlockSpec((block, d), lambda i: (i, 0)),
            pl.BlockSpec((block, 1), lambda i: (i, 0)),
            pl.BlockSpec((1, d), lambda i: (0, 0)),
        ],
        out_specs=pl.BlockSpec((block, d), lambda i: (i, 0)),
        out_shape=jax.ShapeDtypeStruct((n, d), jnp.float32),
    )
    return fin_k(agg0, agg1, g, dinv, b.reshape(1, d))
